# Initial kernel scaffold; baseline (speedup 1.0000x reference)
import jax
import jax.numpy as jnp
from jax.experimental import pallas as pl


def kernel(x, table, W1, b1, gamma, beta, W2, b2):
    raise NotImplementedError("write your pallas kernel here")


# speedup vs baseline: 3.1185x; 3.1185x over previous
import functools

import jax
import jax.numpy as jnp
from jax import lax
from jax.experimental import pallas as pl
from jax.experimental.pallas import tpu as pltpu
from jax.experimental.pallas import tpu_sc as plsc

_B = 16384
_SEQ = 200
_H = 256
_NC = 2
_NS = 16
_NW = _NC * _NS
_RPW = _B // _NW
_C0 = 104
_C1 = 96
_VPH = _H // 16


def _sc_pool_body(x_hbm, p_hbm, out_hbm, idx_v, buf0, buf1, stage, sem0, sem1):
    wid = lax.axis_index("s") * _NC + lax.axis_index("c")
    base = wid * _RPW

    def accum(buf, n, acc):
        def step(g, a):
            return tuple(a[i] + buf[g, pl.ds(i * 16, 16)] for i in range(_VPH))
        return lax.fori_loop(0, n, step, acc)

    def row_body(r, carry):
        b = base + r
        pltpu.sync_copy(x_hbm.at[b], idx_v)
        cp0 = pltpu.async_copy(p_hbm.at[idx_v.at[pl.ds(0, _C0)]], buf0, sem0)
        cp1 = pltpu.async_copy(p_hbm.at[idx_v.at[pl.ds(_C0, _C1)]], buf1, sem1)
        zero = jnp.zeros((16,), jnp.float32)
        acc = tuple(zero for _ in range(_VPH))
        cp0.wait()
        acc = accum(buf0, _C0, acc)
        cp1.wait()
        acc = accum(buf1, _C1, acc)
        for i in range(_VPH):
            stage[pl.ds(i * 16, 16)] = acc[i]
        pltpu.sync_copy(stage, out_hbm.at[b])
        return carry

    lax.fori_loop(0, _RPW, row_body, 0)


def _sc_gather_sum(x, p):
    mesh = plsc.VectorSubcoreMesh(core_axis_name="c", subcore_axis_name="s")
    f = pl.kernel(
        _sc_pool_body,
        out_type=jax.ShapeDtypeStruct((_B, _H), jnp.float32),
        mesh=mesh,
        scratch_types=[
            pltpu.VMEM((_SEQ,), jnp.int32),
            pltpu.VMEM((_C0, _H), jnp.float32),
            pltpu.VMEM((_C1, _H), jnp.float32),
            pltpu.VMEM((_H,), jnp.float32),
            pltpu.SemaphoreType.DMA,
            pltpu.SemaphoreType.DMA,
        ],
    )
    return f(x, p)


def _proj_kernel(t_ref, w_ref, o_ref):
    o_ref[...] = jnp.dot(t_ref[...], w_ref[...],
                         preferred_element_type=jnp.float32)


def _project(table, w1):
    v, d = table.shape
    h = w1.shape[1]
    blk = 2048
    nblk = (v + blk - 1) // blk
    return pl.pallas_call(
        _proj_kernel,
        grid=(nblk,),
        in_specs=[
            pl.BlockSpec((blk, d), lambda i: (i, 0)),
            pl.BlockSpec((d, h), lambda i: (0, 0)),
        ],
        out_specs=pl.BlockSpec((blk, h), lambda i: (i, 0)),
        out_shape=jax.ShapeDtypeStruct((v, h), jnp.float32),
    )(table, w1)


def _stats_kernel(hs_ref, s1_ref, s2_ref):
    @pl.when(pl.program_id(0) == 0)
    def _():
        s1_ref[...] = jnp.zeros_like(s1_ref)
        s2_ref[...] = jnp.zeros_like(s2_ref)
    blk = hs_ref[...]
    s1_ref[...] += jnp.sum(blk, axis=0, keepdims=True)
    s2_ref[...] += jnp.sum(blk * blk, axis=0, keepdims=True)


def _batch_stats(hsum):
    nblk = 32
    blk = _B // nblk
    return pl.pallas_call(
        _stats_kernel,
        grid=(nblk,),
        in_specs=[pl.BlockSpec((blk, _H), lambda i: (i, 0))],
        out_specs=[
            pl.BlockSpec((1, _H), lambda i: (0, 0)),
            pl.BlockSpec((1, _H), lambda i: (0, 0)),
        ],
        out_shape=[
            jax.ShapeDtypeStruct((1, _H), jnp.float32),
            jax.ShapeDtypeStruct((1, _H), jnp.float32),
        ],
    )(hsum)


def _apply_kernel(hs_ref, s1_ref, s2_ref, b1_ref, g_ref, be_ref, w2_ref,
                  b2_ref, o_ref):
    inv_b = 1.0 / _B
    inv_s = 1.0 / _SEQ
    m1 = s1_ref[...] * inv_b
    var_hs = s2_ref[...] * inv_b - m1 * m1
    inv_std = lax.rsqrt(var_hs * (inv_s * inv_s) + 1e-5)
    scale = g_ref[...] * inv_std * inv_s
    hn = (hs_ref[...] - m1) * scale + be_ref[...]
    hr = jnp.maximum(hn, 0.0)
    o_ref[...] = jnp.dot(hr, w2_ref[...],
                         preferred_element_type=jnp.float32) + b2_ref[...]


def _finish(hsum, s1, s2, b1, gamma, beta, w2, b2):
    nblk = 16
    blk = _B // nblk
    nlab = w2.shape[1]
    row = lambda i: (0, 0)
    return pl.pallas_call(
        _apply_kernel,
        grid=(nblk,),
        in_specs=[
            pl.BlockSpec((blk, _H), lambda i: (i, 0)),
            pl.BlockSpec((1, _H), row),
            pl.BlockSpec((1, _H), row),
            pl.BlockSpec((1, _H), row),
            pl.BlockSpec((1, _H), row),
            pl.BlockSpec((1, _H), row),
            pl.BlockSpec((_H, nlab), row),
            pl.BlockSpec((1, nlab), row),
        ],
        out_specs=pl.BlockSpec((blk, nlab), lambda i: (i, 0)),
        out_shape=jax.ShapeDtypeStruct((_B, nlab), jnp.float32),
    )(hsum, s1, s2, b1, gamma, beta, w2, b2)


def kernel(x, table, W1, b1, gamma, beta, W2, b2):
    p = _project(table, W1)
    hsum = _sc_gather_sum(x, p)
    s1, s2 = _batch_stats(hsum)
    return _finish(hsum, s1, s2, b1.reshape(1, _H), gamma.reshape(1, _H),
                   beta.reshape(1, _H), W2, b2.reshape(1, -1))


# speedup vs baseline: 5.9330x; 1.9025x over previous
import functools

import jax
import jax.numpy as jnp
from jax import lax
from jax.experimental import pallas as pl
from jax.experimental.pallas import tpu as pltpu
from jax.experimental.pallas import tpu_sc as plsc

_B = 16384
_SEQ = 200
_H = 256
_NC = 2
_NS = 16
_NW = _NC * _NS
_RPW = _B // _NW
_C0 = 104
_C1 = 96
_VPH = _H // 16


def _sc_pool_body(x_hbm, p_hbm, out_hbm, idx0, idx1, buf00, buf01, buf10,
                  buf11, stage0, stage1, s_idx, sg0, sg1, so0, so1):
    wid = lax.axis_index("s") * _NC + lax.axis_index("c")
    base = wid * _RPW
    sets = (
        (idx0, buf00, buf01, stage0, sg0, so0),
        (idx1, buf10, buf11, stage1, sg1, so1),
    )

    def gather_pair(p):
        idx, b0, b1, _, sg, _ = sets[p]
        c0 = pltpu.make_async_copy(p_hbm.at[idx.at[pl.ds(0, _C0)]], b0, sg)
        c1 = pltpu.make_async_copy(p_hbm.at[idx.at[pl.ds(_C0, _C1)]], b1, sg)
        return c0, c1

    def accum(buf, n, acc):
        def step(g, a):
            return tuple(a[i] + buf[g, pl.ds(i * 16, 16)]
                         for i in range(_VPH))
        return lax.fori_loop(0, n, step, acc)

    def store_wait(p, b):
        _, _, _, stage, _, so = sets[p]
        pltpu.make_async_copy(stage, out_hbm.at[b], so).wait()

    def process(p, b, first_guard):
        idx, b0, b1, stage, sg, so = sets[p]
        c0, c1 = gather_pair(p)
        c0.wait()
        c1.wait()

        @pl.when(b + 2 < _RPW)
        def _():
            pltpu.async_copy(x_hbm.at[base + b + 2], idx, s_idx)
        zero = jnp.zeros((16,), jnp.float32)
        acc = tuple(zero for _ in range(_VPH))
        acc = accum(b0, _C0, acc)
        acc = accum(b1, _C1, acc)

        @pl.when(first_guard)
        def _():
            store_wait(p, base + b)
        for i in range(_VPH):
            stage[pl.ds(i * 16, 16)] = acc[i]
        pltpu.async_copy(stage, out_hbm.at[base + b], so)

    pltpu.sync_copy(x_hbm.at[base], idx0)
    c0, c1 = gather_pair(0)
    c0.start()
    c1.start()
    pltpu.async_copy(x_hbm.at[base + 1], idx1, s_idx)

    def loop_body(r2, carry):
        a = 2 * r2
        pltpu.make_async_copy(x_hbm.at[base + a + 1], idx1, s_idx).wait()
        g0, g1 = gather_pair(1)
        g0.start()
        g1.start()
        process(0, a, r2 >= 1)
        @pl.when(a + 2 < _RPW)
        def _():
            pltpu.make_async_copy(x_hbm.at[base + a + 2], idx0, s_idx).wait()
            n0, n1 = gather_pair(0)
            n0.start()
            n1.start()
        process(1, a + 1, r2 >= 1)
        return carry

    lax.fori_loop(0, _RPW // 2, loop_body, 0)
    store_wait(0, base + _RPW - 2)
    store_wait(1, base + _RPW - 1)


def _sc_gather_sum(x, p):
    mesh = plsc.VectorSubcoreMesh(core_axis_name="c", subcore_axis_name="s")
    f = pl.kernel(
        _sc_pool_body,
        out_type=jax.ShapeDtypeStruct((_B, _H), jnp.float32),
        mesh=mesh,
        scratch_types=[
            pltpu.VMEM((_SEQ,), jnp.int32),
            pltpu.VMEM((_SEQ,), jnp.int32),
            pltpu.VMEM((_C0, _H), jnp.float32),
            pltpu.VMEM((_C1, _H), jnp.float32),
            pltpu.VMEM((_C0, _H), jnp.float32),
            pltpu.VMEM((_C1, _H), jnp.float32),
            pltpu.VMEM((_H,), jnp.float32),
            pltpu.VMEM((_H,), jnp.float32),
            pltpu.SemaphoreType.DMA,
            pltpu.SemaphoreType.DMA,
            pltpu.SemaphoreType.DMA,
            pltpu.SemaphoreType.DMA,
            pltpu.SemaphoreType.DMA,
        ],
    )
    return f(x, p)


def _proj_kernel(t_ref, w_ref, o_ref):
    o_ref[...] = jnp.dot(t_ref[...], w_ref[...],
                         preferred_element_type=jnp.float32)


def _project(table, w1):
    v, d = table.shape
    h = w1.shape[1]
    blk = 2048
    nblk = (v + blk - 1) // blk
    return pl.pallas_call(
        _proj_kernel,
        grid=(nblk,),
        in_specs=[
            pl.BlockSpec((blk, d), lambda i: (i, 0)),
            pl.BlockSpec((d, h), lambda i: (0, 0)),
        ],
        out_specs=pl.BlockSpec((blk, h), lambda i: (i, 0)),
        out_shape=jax.ShapeDtypeStruct((v, h), jnp.float32),
    )(table, w1)


def _stats_kernel(hs_ref, s1_ref, s2_ref):
    @pl.when(pl.program_id(0) == 0)
    def _():
        s1_ref[...] = jnp.zeros_like(s1_ref)
        s2_ref[...] = jnp.zeros_like(s2_ref)
    blk = hs_ref[...]
    s1_ref[...] += jnp.sum(blk, axis=0, keepdims=True)
    s2_ref[...] += jnp.sum(blk * blk, axis=0, keepdims=True)


def _batch_stats(hsum):
    nblk = 32
    blk = _B // nblk
    return pl.pallas_call(
        _stats_kernel,
        grid=(nblk,),
        in_specs=[pl.BlockSpec((blk, _H), lambda i: (i, 0))],
        out_specs=[
            pl.BlockSpec((1, _H), lambda i: (0, 0)),
            pl.BlockSpec((1, _H), lambda i: (0, 0)),
        ],
        out_shape=[
            jax.ShapeDtypeStruct((1, _H), jnp.float32),
            jax.ShapeDtypeStruct((1, _H), jnp.float32),
        ],
    )(hsum)


def _apply_kernel(hs_ref, s1_ref, s2_ref, b1_ref, g_ref, be_ref, w2_ref,
                  b2_ref, o_ref):
    inv_b = 1.0 / _B
    inv_s = 1.0 / _SEQ
    m1 = s1_ref[...] * inv_b
    var_hs = s2_ref[...] * inv_b - m1 * m1
    inv_std = lax.rsqrt(var_hs * (inv_s * inv_s) + 1e-5)
    scale = g_ref[...] * inv_std * inv_s
    hn = (hs_ref[...] - m1) * scale + be_ref[...]
    hr = jnp.maximum(hn, 0.0)
    o_ref[...] = jnp.dot(hr, w2_ref[...],
                         preferred_element_type=jnp.float32) + b2_ref[...]


def _finish(hsum, s1, s2, b1, gamma, beta, w2, b2):
    nblk = 16
    blk = _B // nblk
    nlab = w2.shape[1]
    row = lambda i: (0, 0)
    return pl.pallas_call(
        _apply_kernel,
        grid=(nblk,),
        in_specs=[
            pl.BlockSpec((blk, _H), lambda i: (i, 0)),
            pl.BlockSpec((1, _H), row),
            pl.BlockSpec((1, _H), row),
            pl.BlockSpec((1, _H), row),
            pl.BlockSpec((1, _H), row),
            pl.BlockSpec((1, _H), row),
            pl.BlockSpec((_H, nlab), row),
            pl.BlockSpec((1, nlab), row),
        ],
        out_specs=pl.BlockSpec((blk, nlab), lambda i: (i, 0)),
        out_shape=jax.ShapeDtypeStruct((_B, nlab), jnp.float32),
    )(hsum, s1, s2, b1, gamma, beta, w2, b2)


def kernel(x, table, W1, b1, gamma, beta, W2, b2):
    p = _project(table, W1)
    hsum = _sc_gather_sum(x, p)
    s1, s2 = _batch_stats(hsum)
    return _finish(hsum, s1, s2, b1.reshape(1, _H), gamma.reshape(1, _H),
                   beta.reshape(1, _H), W2, b2.reshape(1, -1))
